# baseline (speedup 1.0000x reference)
import jax
import jax.numpy as jnp
from jax import lax
from jax.experimental import pallas as pl
from jax.experimental.pallas import tpu as pltpu
from jax.experimental.pallas import tpu_sc as plsc

N_CORES = 2
N_SUBCORES = 16
N_WORKERS = N_CORES * N_SUBCORES
LANES = 16

BATCH = 16384
DIM = 128
B_W = BATCH // N_WORKERS
CHUNK = 64
N_CHUNKS = B_W // CHUNK
NBUF = 4
AHEAD = NBUF - 1
GROUPS = CHUNK // LANES
SLICES = DIM // LANES
TPAD = 24


def _sc_body(h_hbm, r_hbm, t_hbm, ent_hbm, rel_hbm, out_hbm,
             h_idx, r_idx, t_idx, out_v, tr, *rest):
    bufs = tuple((rest[3 * i], rest[3 * i + 1], rest[3 * i + 2],
                  rest[3 * NBUF + i]) for i in range(NBUF))

    wid = lax.axis_index("s") * N_CORES + lax.axis_index("c")
    base = pl.multiple_of(wid * B_W, B_W)

    isem = bufs[0][3]
    di0 = pltpu.async_copy(h_hbm.at[pl.ds(base, B_W)], h_idx, isem)
    di1 = pltpu.async_copy(r_hbm.at[pl.ds(base, B_W)], r_idx, isem)
    di2 = pltpu.async_copy(t_hbm.at[pl.ds(base, B_W)], t_idx, isem)
    di0.wait()
    di1.wait()
    di2.wait()

    def start(c):
        hb, rb, tb, sem = bufs[c % NBUF]
        lo = c * CHUNK
        return (
            pltpu.async_copy(ent_hbm.at[h_idx.at[pl.ds(lo, CHUNK)]], hb, sem),
            pltpu.async_copy(rel_hbm.at[r_idx.at[pl.ds(lo, CHUNK)]], rb, sem),
            pltpu.async_copy(ent_hbm.at[t_idx.at[pl.ds(lo, CHUNK)]], tb, sem),
        )

    iota = lax.iota(jnp.int32, LANES)

    def compute(c):
        hb, rb, tb, _ = bufs[c % NBUF]

        def group(g, carry):
            row0 = pl.multiple_of(g * LANES, LANES)
            for j in range(LANES):
                row = row0 + j
                acc = jnp.zeros((LANES,), jnp.float32)
                for s in range(SLICES):
                    sl = pl.ds(s * LANES, LANES)
                    acc = acc + jnp.abs(hb[row, sl] + rb[row, sl] - tb[row, sl])
                tr[pl.ds(j * TPAD, LANES)] = acc
            tot = plsc.load_gather(tr, [iota * TPAD])
            for i in range(1, LANES):
                tot = tot + plsc.load_gather(tr, [iota * TPAD + i])
            out_v[pl.ds(c * CHUNK + row0, LANES)] = -tot
            return carry

        lax.fori_loop(0, GROUPS, group, 0)

    descs = [start(c) for c in range(AHEAD)]
    for c in range(N_CHUNKS):
        if c + AHEAD < N_CHUNKS:
            descs.append(start(c + AHEAD))
        for d in descs[c]:
            d.wait()
        compute(c)

    pltpu.sync_copy(out_v, out_hbm.at[pl.ds(base, B_W)])


def _make_kernel():
    mesh = plsc.VectorSubcoreMesh(core_axis_name="c", subcore_axis_name="s",
                                  num_cores=N_CORES, num_subcores=N_SUBCORES)
    return pl.kernel(
        _sc_body,
        out_type=jax.ShapeDtypeStruct((BATCH,), jnp.float32),
        mesh=mesh,
        compiler_params=pltpu.CompilerParams(needs_layout_passes=False),
        scratch_types=(
            [pltpu.VMEM((B_W,), jnp.int32)] * 3
            + [pltpu.VMEM((B_W,), jnp.float32),
               pltpu.VMEM((LANES * TPAD,), jnp.float32)]
            + [pltpu.VMEM((CHUNK, DIM), jnp.float32)] * (3 * NBUF)
            + [pltpu.SemaphoreType.DMA] * NBUF
        ),
    )


@jax.jit
def kernel(h, r, t, entity_embedding, relation_embedding):
    fn = _make_kernel()
    return fn(h.astype(jnp.int32), r.astype(jnp.int32), t.astype(jnp.int32),
              entity_embedding, relation_embedding)


# speedup vs baseline: 1.2151x; 1.2151x over previous
import jax
import jax.numpy as jnp
from jax import lax
from jax.experimental import pallas as pl
from jax.experimental.pallas import tpu as pltpu
from jax.experimental.pallas import tpu_sc as plsc

N_CORES = 2
N_SUBCORES = 16
N_WORKERS = N_CORES * N_SUBCORES
LANES = 16

N_RELATION = 1000
BATCH = 16384
DIM = 128
B_W = BATCH // N_WORKERS
CHUNK = 128
N_CHUNKS = B_W // CHUNK
GROUPS = CHUNK // LANES
SLICES = DIM // LANES
TPAD = 24


def _sc_body(h_hbm, r_hbm, t_hbm, ent_hbm, rel_hbm, out_hbm,
             h_idx, r_idx, t_idx,
             hb0, rb0, tb0, hb1, rb1, tb1,
             out_v, tr, sem0, sem1):
    sid = lax.axis_index("s")
    wid = sid * N_CORES + lax.axis_index("c")
    base = pl.multiple_of(wid * B_W, B_W)

    isem = sem0
    di0 = pltpu.async_copy(h_hbm.at[pl.ds(base, B_W)], h_idx, isem)
    di1 = pltpu.async_copy(r_hbm.at[pl.ds(base, B_W)], r_idx, isem)
    di2 = pltpu.async_copy(t_hbm.at[pl.ds(base, B_W)], t_idx, isem)
    di0.wait()
    di1.wait()
    di2.wait()

    bufs = ((hb0, rb0, tb0, sem0), (hb1, rb1, tb1, sem1))

    def start(c):
        hb, rb, tb, sem = bufs[c % 2]
        lo = c * CHUNK
        return (
            pltpu.async_copy(ent_hbm.at[h_idx.at[pl.ds(lo, CHUNK)]], hb, sem),
            pltpu.async_copy(rel_hbm.at[r_idx.at[pl.ds(lo, CHUNK)]], rb, sem),
            pltpu.async_copy(ent_hbm.at[t_idx.at[pl.ds(lo, CHUNK)]], tb, sem),
        )

    iota = lax.iota(jnp.int32, LANES)

    def compute(c):
        hb, rb, tb, _ = bufs[c % 2]

        def group(g, carry):
            row0 = pl.multiple_of(g * LANES, LANES)

            def one_row(j, carry2):
                row = row0 + j
                acc = jnp.zeros((LANES,), jnp.float32)
                for s in range(SLICES):
                    sl = pl.ds(s * LANES, LANES)
                    acc = acc + jnp.abs(hb[row, sl] + rb[row, sl] - tb[row, sl])
                tr[pl.ds(j * TPAD, LANES)] = acc
                return carry2

            lax.fori_loop(0, LANES, one_row, 0)
            tot = plsc.load_gather(tr, [iota * TPAD])
            for i in range(1, LANES):
                tot = tot + plsc.load_gather(tr, [iota * TPAD + i])
            out_v[pl.ds(c * CHUNK + row0, LANES)] = -tot
            return carry

        lax.fori_loop(0, GROUPS, group, 0)

    descs = [start(0)]
    for c in range(N_CHUNKS):
        if c + 1 < N_CHUNKS:
            descs.append(start(c + 1))
        for d in descs[c]:
            d.wait()
        compute(c)

    pltpu.sync_copy(out_v, out_hbm.at[pl.ds(base, B_W)])


def _make_kernel():
    mesh = plsc.VectorSubcoreMesh(core_axis_name="c", subcore_axis_name="s",
                                  num_cores=N_CORES, num_subcores=N_SUBCORES)
    return pl.kernel(
        _sc_body,
        out_type=jax.ShapeDtypeStruct((BATCH,), jnp.float32),
        mesh=mesh,
        compiler_params=pltpu.CompilerParams(needs_layout_passes=False),
        scratch_types=(
            [pltpu.VMEM((B_W,), jnp.int32)] * 3
            + [pltpu.VMEM((CHUNK, DIM), jnp.float32)] * 6
            + [pltpu.VMEM((B_W,), jnp.float32),
               pltpu.VMEM((LANES * TPAD,), jnp.float32)]
            + [pltpu.SemaphoreType.DMA] * 2
        ),
    )


@jax.jit
def kernel(h, r, t, entity_embedding, relation_embedding):
    fn = _make_kernel()
    return fn(h.astype(jnp.int32), r.astype(jnp.int32), t.astype(jnp.int32),
              entity_embedding, relation_embedding)


# speedup vs baseline: 1.2210x; 1.0048x over previous
import jax
import jax.numpy as jnp
from jax import lax
from jax.experimental import pallas as pl
from jax.experimental.pallas import tpu as pltpu
from jax.experimental.pallas import tpu_sc as plsc

N_CORES = 2
N_SUBCORES = 16
N_WORKERS = N_CORES * N_SUBCORES
LANES = 16

N_RELATION = 1000
BATCH = 16384
DIM = 128
B_W = BATCH // N_WORKERS
CHUNK = 128
N_CHUNKS = B_W // CHUNK
GROUPS = CHUNK // LANES
SLICES = DIM // LANES
TPAD = 24


def _sc_body(h_hbm, r_hbm, t_hbm, ent_hbm, rel_hbm, out_hbm,
             h_idx, r_idx, t_idx,
             hb0, rb0, tb0, hb1, rb1, tb1,
             out_v, tr, sem0, sem1, sem2):
    sid = lax.axis_index("s")
    wid = sid * N_CORES + lax.axis_index("c")
    base = pl.multiple_of(wid * B_W, B_W)

    bufs = ((hb0, rb0, tb0, sem0), (hb1, rb1, tb1, sem1))

    def start(c):
        hb, rb, tb, sem = bufs[c % 2]
        lo = c * CHUNK
        return (
            pltpu.async_copy(ent_hbm.at[h_idx.at[pl.ds(lo, CHUNK)]], hb, sem),
            pltpu.async_copy(rel_hbm.at[r_idx.at[pl.ds(lo, CHUNK)]], rb, sem),
            pltpu.async_copy(ent_hbm.at[t_idx.at[pl.ds(lo, CHUNK)]], tb, sem),
        )

    hb_0, rb_0, tb_0, gsem0 = bufs[0]
    isem = sem1
    di0 = pltpu.async_copy(h_hbm.at[pl.ds(base, B_W)], h_idx, isem)
    di1 = pltpu.async_copy(r_hbm.at[pl.ds(base, B_W)], r_idx, isem)
    di2 = pltpu.async_copy(t_hbm.at[pl.ds(base, B_W)], t_idx, isem)
    di0.wait()
    g0 = pltpu.async_copy(ent_hbm.at[h_idx.at[pl.ds(0, CHUNK)]], hb_0, gsem0)
    di1.wait()
    g1 = pltpu.async_copy(rel_hbm.at[r_idx.at[pl.ds(0, CHUNK)]], rb_0, gsem0)
    di2.wait()
    g2 = pltpu.async_copy(ent_hbm.at[t_idx.at[pl.ds(0, CHUNK)]], tb_0, gsem0)

    iota = lax.iota(jnp.int32, LANES)

    def compute(c):
        hb, rb, tb, _ = bufs[c % 2]

        def group(g, carry):
            row0 = pl.multiple_of(g * LANES, LANES)

            def one_row(j, carry2):
                row = row0 + j
                acc = jnp.zeros((LANES,), jnp.float32)
                for s in range(SLICES):
                    sl = pl.ds(s * LANES, LANES)
                    acc = acc + jnp.abs(hb[row, sl] + rb[row, sl] - tb[row, sl])
                tr[pl.ds(j * TPAD, LANES)] = acc
                return carry2

            lax.fori_loop(0, LANES, one_row, 0)
            tot = plsc.load_gather(tr, [iota * TPAD])
            for i in range(1, LANES):
                tot = tot + plsc.load_gather(tr, [iota * TPAD + i])
            out_v[pl.ds(c * CHUNK + row0, LANES)] = -tot
            return carry

        lax.fori_loop(0, GROUPS, group, 0)

    descs = [(g0, g1, g2)]
    osem = sem2
    odescs = []
    for c in range(N_CHUNKS):
        if c + 1 < N_CHUNKS:
            descs.append(start(c + 1))
        for d in descs[c]:
            d.wait()
        compute(c)
        odescs.append(pltpu.async_copy(
            out_v.at[pl.ds(c * CHUNK, CHUNK)],
            out_hbm.at[pl.ds(base + c * CHUNK, CHUNK)], osem))
    for d in odescs:
        d.wait()


def _make_kernel():
    mesh = plsc.VectorSubcoreMesh(core_axis_name="c", subcore_axis_name="s",
                                  num_cores=N_CORES, num_subcores=N_SUBCORES)
    return pl.kernel(
        _sc_body,
        out_type=jax.ShapeDtypeStruct((BATCH,), jnp.float32),
        mesh=mesh,
        compiler_params=pltpu.CompilerParams(needs_layout_passes=False),
        scratch_types=(
            [pltpu.VMEM((B_W,), jnp.int32)] * 3
            + [pltpu.VMEM((CHUNK, DIM), jnp.float32)] * 6
            + [pltpu.VMEM((B_W,), jnp.float32),
               pltpu.VMEM((LANES * TPAD,), jnp.float32)]
            + [pltpu.SemaphoreType.DMA] * 3
        ),
    )


@jax.jit
def kernel(h, r, t, entity_embedding, relation_embedding):
    fn = _make_kernel()
    return fn(h.astype(jnp.int32), r.astype(jnp.int32), t.astype(jnp.int32),
              entity_embedding, relation_embedding)
